# baseline (speedup 1.0000x reference)
import jax
import jax.numpy as jnp
from jax.experimental import pallas as pl
from jax.experimental.pallas import tpu as pltpu

_N = 100000
_D = 128
_R = 2000
_NBLK = _N // _R


def _body(z_ref, zp_ref, zn_ref, x_ref, xh_ref, acc_ref):
    i = pl.program_id(0)

    @pl.when(i == 0)
    def _init():
        acc_ref[0] = 0.0
        acc_ref[1] = 0.0
        acc_ref[2] = 0.0

    z = z_ref[...]
    ones_row = jnp.ones((1, _D), dtype=jnp.float32)
    dnums = (((1,), (1,)), ((), ()))
    pdot = jax.lax.dot_general(ones_row, z * zp_ref[...], dnums,
                               preferred_element_type=jnp.float32)
    ndot = jax.lax.dot_general(ones_row, z * zn_ref[...], dnums,
                               preferred_element_type=jnp.float32)
    pos_part = jnp.sum(jax.nn.log_sigmoid(pdot))
    neg_part = jnp.sum(jax.nn.log_sigmoid(-ndot))
    diff = x_ref[...] - xh_ref[...]
    mse_part = jnp.sum(diff * diff)
    acc_ref[0] += pos_part
    acc_ref[1] += neg_part
    acc_ref[2] += mse_part


def kernel(out, x_full, xhat_full, lamb):
    row_spec = pl.BlockSpec((_R, _D), lambda i: (i, 0))
    sums = pl.pallas_call(
        _body,
        grid=(_NBLK,),
        in_specs=[
            pl.BlockSpec((_R, _D), lambda i: (i, 0)),
            pl.BlockSpec((_R, _D), lambda i: (i + _NBLK, 0)),
            pl.BlockSpec((_R, _D), lambda i: (i + 2 * _NBLK, 0)),
            row_spec,
            row_spec,
        ],
        out_specs=pl.BlockSpec(memory_space=pltpu.SMEM),
        out_shape=jax.ShapeDtypeStruct((3,), jnp.float32),
    )(out, out, out, x_full, xhat_full)

    lamb = jnp.clip(lamb, 1e-08, 1.0 - 1e-08)
    pos_loss = sums[0] / _N
    neg_loss = sums[1] / _N
    mse = sums[2] / (_N * _D)
    return lamb * mse + (1.0 - lamb) * (-pos_loss - neg_loss)


# speedup vs baseline: 1.0662x; 1.0662x over previous
import jax
import jax.numpy as jnp
from jax.experimental import pallas as pl
from jax.experimental.pallas import tpu as pltpu

_N = 100000
_D = 128
_R = 4000
_NBLK = _N // _R


def _body(z_ref, zp_ref, zn_ref, x_ref, xh_ref, acc_ref):
    i = pl.program_id(0)

    @pl.when(i == 0)
    def _init():
        acc_ref[0] = 0.0
        acc_ref[1] = 0.0
        acc_ref[2] = 0.0

    z = z_ref[...]
    ones_row = jnp.ones((1, _D), dtype=jnp.float32)
    dnums = (((1,), (1,)), ((), ()))
    pdot = jax.lax.dot_general(ones_row, z * zp_ref[...], dnums,
                               preferred_element_type=jnp.float32)
    ndot = jax.lax.dot_general(ones_row, z * zn_ref[...], dnums,
                               preferred_element_type=jnp.float32)
    pos_part = jnp.sum(jax.nn.log_sigmoid(pdot))
    neg_part = jnp.sum(jax.nn.log_sigmoid(-ndot))
    diff = x_ref[...] - xh_ref[...]
    mse_part = jnp.sum(diff * diff)
    acc_ref[0] += pos_part
    acc_ref[1] += neg_part
    acc_ref[2] += mse_part


def kernel(out, x_full, xhat_full, lamb):
    row_spec = pl.BlockSpec((_R, _D), lambda i: (i, 0))
    sums = pl.pallas_call(
        _body,
        grid=(_NBLK,),
        in_specs=[
            pl.BlockSpec((_R, _D), lambda i: (i, 0)),
            pl.BlockSpec((_R, _D), lambda i: (i + _NBLK, 0)),
            pl.BlockSpec((_R, _D), lambda i: (i + 2 * _NBLK, 0)),
            row_spec,
            row_spec,
        ],
        out_specs=pl.BlockSpec(memory_space=pltpu.SMEM),
        out_shape=jax.ShapeDtypeStruct((3,), jnp.float32),
    )(out, out, out, x_full, xhat_full)

    lamb = jnp.clip(lamb, 1e-08, 1.0 - 1e-08)
    pos_loss = sums[0] / _N
    neg_loss = sums[1] / _N
    mse = sums[2] / (_N * _D)
    return lamb * mse + (1.0 - lamb) * (-pos_loss - neg_loss)
